# baseline (speedup 1.0000x reference)
import functools

import numpy as np
import jax
import jax.numpy as jnp
from jax import lax
from jax.experimental import pallas as pl
from jax.experimental.pallas import tpu as pltpu
from jax.experimental.pallas import tpu_sc as plsc

D_MODEL = 64
MAX_LEN = 200

NC = 2
NS = 16
NW = NC * NS
IDX_MINOR = 128


def _pos_encoding_np(position, d_model):
    angle_rads = np.arange(position)[:, np.newaxis] / np.power(
        10000, 2 * (np.arange(d_model)[np.newaxis, :] // 2) / np.float32(d_model))
    angle_rads[:, 0::2] = np.sin(angle_rads[:, 0::2])
    angle_rads[:, 1::2] = np.cos(angle_rads[:, 1::2])
    return angle_rads.astype(np.float32)


def _make_sc_kernel(n_rows, chunk_rows, n_chunks):
    rows_per_w = chunk_rows * n_chunks
    cpc = chunk_rows // IDX_MINOR
    mesh = plsc.VectorSubcoreMesh(
        core_axis_name="c", subcore_axis_name="s",
        num_cores=NC, num_subcores=NS)

    @functools.partial(
        pl.kernel,
        mesh=mesh,
        out_type=jax.ShapeDtypeStruct((n_rows, 2 * D_MODEL), jnp.float32),
        scratch_types=[
            pltpu.VMEM((n_chunks, chunk_rows), jnp.int32),
            pltpu.VMEM((chunk_rows, D_MODEL), jnp.float32),
            pltpu.VMEM((chunk_rows, D_MODEL), jnp.float32),
            pltpu.SemaphoreType.DMA,
            pltpu.SemaphoreType.DMA,
            pltpu.SemaphoreType.DMA,
            pltpu.SemaphoreType.DMA,
            pltpu.SemaphoreType.DMA,
        ],
        compiler_params=pltpu.CompilerParams(use_tc_tiling_on_sc=False),
    )
    def sc_kernel(idx_hbm, table_hbm, pos_hbm, out_hbm,
                  idx_v, buf0, buf1, gsem, p0sem, p1sem, w0sem, w1sem):
        wid = lax.axis_index("s") * NC + lax.axis_index("c")
        wbase = wid * rows_per_w
        bufs = [buf0, buf1]
        psems = [p0sem, p1sem]
        wsems = [w0sem, w1sem]

        pltpu.sync_copy(idx_hbm.at[wid], idx_v)

        def prefill(c, b):
            p0 = (c * chunk_rows) % MAX_LEN
            return pltpu.async_copy(
                pos_hbm.at[pl.ds(p0, chunk_rows)], bufs[b], psems[b])

        wb_descs = [None, None]
        pf_descs = [None, None]
        pf_descs[0] = prefill(0, 0)

        for c in range(n_chunks):
            b = c & 1
            pf_descs[b].wait()
            gd = [pltpu.async_copy(
                table_hbm.at[idx_v.at[c]],
                bufs[b], gsem, add=True)]
            if c + 1 < n_chunks:
                nb = b ^ 1
                if wb_descs[nb] is not None:
                    wb_descs[nb].wait()
                pf_descs[nb] = prefill(c + 1, nb)
            for d in gd:
                d.wait()
            wb_descs[b] = pltpu.async_copy(
                bufs[b],
                out_hbm.at[pl.ds(wbase + c * chunk_rows, chunk_rows),
                           pl.ds(0, D_MODEL)],
                wsems[b])

        for d in wb_descs:
            if d is not None:
                d.wait()

    return sc_kernel


@jax.jit
def kernel(x, table):
    batch, seq_len = x.shape
    n_rows = batch * seq_len
    rows_per_w = n_rows // NW
    chunk_rows = 640
    n_chunks = rows_per_w // chunk_rows

    pos = jnp.asarray(np.tile(_pos_encoding_np(MAX_LEN, D_MODEL), (4, 1)))
    idx = x.reshape(NW, rows_per_w // chunk_rows, chunk_rows).astype(jnp.int32)

    one = (jnp.minimum(x[0, 0], 0) + 1).astype(jnp.float32)
    table_l = table * one

    sc_kernel = _make_sc_kernel(n_rows, chunk_rows, n_chunks)
    out = sc_kernel(idx, table_l, pos)
    return out[:, :D_MODEL].reshape(batch, seq_len, D_MODEL)


# speedup vs baseline: 1.4066x; 1.4066x over previous
import functools

import numpy as np
import jax
import jax.numpy as jnp
from jax import lax
from jax.experimental import pallas as pl
from jax.experimental.pallas import tpu as pltpu
from jax.experimental.pallas import tpu_sc as plsc
from jax.experimental.layout import Format, Layout

D_MODEL = 64
MAX_LEN = 200

NC = 2
NS = 16
NW = NC * NS
IDX_MINOR = 128


def _pos_encoding_np(position, d_model):
    angle_rads = np.arange(position)[:, np.newaxis] / np.power(
        10000, 2 * (np.arange(d_model)[np.newaxis, :] // 2) / np.float32(d_model))
    angle_rads[:, 0::2] = np.sin(angle_rads[:, 0::2])
    angle_rads[:, 1::2] = np.cos(angle_rads[:, 1::2])
    return angle_rads.astype(np.float32)


def _make_sc_kernel(n_rows, chunk_rows, n_chunks):
    rows_per_w = chunk_rows * n_chunks
    cpc = chunk_rows // IDX_MINOR
    mesh = plsc.VectorSubcoreMesh(
        core_axis_name="c", subcore_axis_name="s",
        num_cores=NC, num_subcores=NS)

    @functools.partial(
        pl.kernel,
        mesh=mesh,
        out_type=jax.ShapeDtypeStruct((n_rows, 2 * D_MODEL), jnp.float32),
        scratch_types=[
            pltpu.VMEM((n_chunks, chunk_rows), jnp.int32),
            pltpu.VMEM((chunk_rows, D_MODEL), jnp.float32),
            pltpu.VMEM((chunk_rows, D_MODEL), jnp.float32),
            pltpu.SemaphoreType.DMA,
            pltpu.SemaphoreType.DMA,
            pltpu.SemaphoreType.DMA,
            pltpu.SemaphoreType.DMA,
            pltpu.SemaphoreType.DMA,
        ],
        compiler_params=pltpu.CompilerParams(
            use_tc_tiling_on_sc=False, needs_layout_passes=False),
    )
    def sc_kernel(idx_hbm, table_hbm, pos_hbm, out_hbm,
                  idx_v, buf0, buf1, gsem, p0sem, p1sem, w0sem, w1sem):
        wid = lax.axis_index("s") * NC + lax.axis_index("c")
        wbase = wid * rows_per_w
        bufs = [buf0, buf1]
        psems = [p0sem, p1sem]
        wsems = [w0sem, w1sem]

        pltpu.sync_copy(idx_hbm.at[wid], idx_v)

        def prefill(c, b):
            p0 = (c * chunk_rows) % MAX_LEN
            return pltpu.async_copy(
                pos_hbm.at[pl.ds(p0, chunk_rows)], bufs[b], psems[b])

        wb_descs = [None, None]
        pf_descs = [None, None]
        pf_descs[0] = prefill(0, 0)

        for c in range(n_chunks):
            b = c & 1
            pf_descs[b].wait()
            gd = [pltpu.async_copy(
                table_hbm.at[idx_v.at[c]],
                bufs[b], gsem, add=True)]
            if c + 1 < n_chunks:
                nb = b ^ 1
                if wb_descs[nb] is not None:
                    wb_descs[nb].wait()
                pf_descs[nb] = prefill(c + 1, nb)
            for d in gd:
                d.wait()
            wb_descs[b] = pltpu.async_copy(
                bufs[b],
                out_hbm.at[pl.ds(wbase + c * chunk_rows, chunk_rows),
                           pl.ds(0, D_MODEL)],
                wsems[b])

        for d in wb_descs:
            if d is not None:
                d.wait()

    return sc_kernel


@jax.jit
def kernel(x, table):
    batch, seq_len = x.shape
    n_rows = batch * seq_len
    rows_per_w = n_rows // NW
    chunk_rows = 640
    n_chunks = rows_per_w // chunk_rows

    pos = jnp.asarray(np.tile(_pos_encoding_np(MAX_LEN, D_MODEL), (4, 1)))
    idx = x.reshape(NW, rows_per_w // chunk_rows, chunk_rows).astype(jnp.int32)

    table_t8 = jax.device_put(
        table, Format(Layout(major_to_minor=(0, 1), tiling=((8,),)),
                      jax.sharding.SingleDeviceSharding(jax.devices()[0])))

    sc_kernel = _make_sc_kernel(n_rows, chunk_rows, n_chunks)
    out = sc_kernel(idx, table_t8, pos)
    return out[:, :D_MODEL].reshape(batch, seq_len, D_MODEL)


# speedup vs baseline: 1.5279x; 1.0862x over previous
import functools

import numpy as np
import jax
import jax.numpy as jnp
from jax import lax
from jax.experimental import pallas as pl
from jax.experimental.pallas import tpu as pltpu
from jax.experimental.pallas import tpu_sc as plsc

D_MODEL = 64
MAX_LEN = 200
POS_REP = 4

NC = 2
NS = 16
NW = NC * NS


def _pos_encoding_np(position, d_model):
    angle_rads = np.arange(position)[:, np.newaxis] / np.power(
        10000, 2 * (np.arange(d_model)[np.newaxis, :] // 2) / np.float32(d_model))
    angle_rads[:, 0::2] = np.sin(angle_rads[:, 0::2])
    angle_rads[:, 1::2] = np.cos(angle_rads[:, 1::2])
    return angle_rads.astype(np.float32)


def _make_sc_kernel(n_rows, chunk_rows, n_chunks):
    rows_per_w = chunk_rows * n_chunks
    pos_rows = MAX_LEN * POS_REP
    mesh = plsc.VectorSubcoreMesh(
        core_axis_name="c", subcore_axis_name="s",
        num_cores=NC, num_subcores=NS)

    @functools.partial(
        pl.kernel,
        mesh=mesh,
        out_type=jax.ShapeDtypeStruct((n_rows, 2 * D_MODEL), jnp.float32),
        scratch_types=[
            pltpu.VMEM((n_chunks, chunk_rows), jnp.int32),
            pltpu.VMEM((chunk_rows, D_MODEL), jnp.float32),
            pltpu.VMEM((chunk_rows, D_MODEL), jnp.float32),
            pltpu.VMEM_SHARED((pos_rows, D_MODEL), jnp.float32),
            pltpu.SemaphoreType.DMA,
            pltpu.SemaphoreType.DMA,
            pltpu.SemaphoreType.DMA,
            pltpu.SemaphoreType.DMA,
            pltpu.SemaphoreType.DMA,
        ],
        compiler_params=pltpu.CompilerParams(use_tc_tiling_on_sc=False),
    )
    def sc_kernel(idx_hbm, table_hbm, pos_hbm, out_hbm,
                  idx_v, buf0, buf1, spos, gsem, p0sem, p1sem, w0sem, w1sem):
        sid = lax.axis_index("s")
        wid = sid * NC + lax.axis_index("c")
        wbase = wid * rows_per_w
        bufs = [buf0, buf1]
        psems = [p0sem, p1sem]
        wsems = [w0sem, w1sem]

        @pl.when(sid == 0)
        def _():
            pltpu.sync_copy(pos_hbm, spos)

        plsc.subcore_barrier()

        pltpu.sync_copy(idx_hbm.at[wid], idx_v)

        def prefill(b):
            return pltpu.async_copy(
                spos.at[pl.ds(0, chunk_rows)], bufs[b], psems[b])

        wb_descs = [None, None]
        pf_descs = [None, None]
        pf_descs[0] = prefill(0)

        for c in range(n_chunks):
            b = c & 1
            pf_descs[b].wait()
            gd = pltpu.async_copy(
                table_hbm.at[idx_v.at[c]], bufs[b], gsem, add=True)
            if c + 1 < n_chunks:
                nb = b ^ 1
                if wb_descs[nb] is not None:
                    wb_descs[nb].wait()
                pf_descs[nb] = prefill(nb)
            gd.wait()
            wb_descs[b] = pltpu.async_copy(
                bufs[b],
                out_hbm.at[pl.ds(wbase + c * chunk_rows, chunk_rows),
                           pl.ds(0, D_MODEL)],
                wsems[b])

        for d in wb_descs:
            if d is not None:
                d.wait()

    return sc_kernel


@jax.jit
def kernel(x, table):
    batch, seq_len = x.shape
    n_rows = batch * seq_len
    rows_per_w = n_rows // NW
    chunk_rows = 800
    n_chunks = rows_per_w // chunk_rows

    pos = jnp.asarray(
        np.tile(_pos_encoding_np(MAX_LEN, D_MODEL), (POS_REP, 1)))
    idx = x.reshape(NW, n_chunks, chunk_rows).astype(jnp.int32)

    sc_kernel = _make_sc_kernel(n_rows, chunk_rows, n_chunks)
    out = sc_kernel(idx, table, pos)
    return out[:, :D_MODEL].reshape(batch, seq_len, D_MODEL)


# speedup vs baseline: 1.5319x; 1.0027x over previous
import functools

import numpy as np
import jax
import jax.numpy as jnp
from jax import lax
from jax.experimental import pallas as pl
from jax.experimental.pallas import tpu as pltpu
from jax.experimental.pallas import tpu_sc as plsc

D_MODEL = 64
MAX_LEN = 200
POS_REP = 4

NC = 2
NS = 16
NW = NC * NS


def _pos_encoding_np(position, d_model):
    angle_rads = np.arange(position)[:, np.newaxis] / np.power(
        10000, 2 * (np.arange(d_model)[np.newaxis, :] // 2) / np.float32(d_model))
    angle_rads[:, 0::2] = np.sin(angle_rads[:, 0::2])
    angle_rads[:, 1::2] = np.cos(angle_rads[:, 1::2])
    return angle_rads.astype(np.float32)


def _make_sc_kernel(n_rows, chunk_rows, n_chunks):
    rows_per_w = chunk_rows * n_chunks
    pos_rows = MAX_LEN * POS_REP
    mesh = plsc.VectorSubcoreMesh(
        core_axis_name="c", subcore_axis_name="s",
        num_cores=NC, num_subcores=NS)

    @functools.partial(
        pl.kernel,
        mesh=mesh,
        out_type=jax.ShapeDtypeStruct((n_rows, 2 * D_MODEL), jnp.float32),
        scratch_types=[
            pltpu.VMEM((n_chunks, chunk_rows), jnp.int32),
            pltpu.VMEM((chunk_rows, D_MODEL), jnp.float32),
            pltpu.VMEM((chunk_rows, D_MODEL), jnp.float32),
            pltpu.VMEM((chunk_rows, D_MODEL), jnp.float32),
            pltpu.VMEM((chunk_rows, D_MODEL), jnp.float32),
            pltpu.VMEM_SHARED((pos_rows, D_MODEL), jnp.float32),
            pltpu.SemaphoreType.DMA,
            pltpu.SemaphoreType.DMA,
            pltpu.SemaphoreType.DMA,
            pltpu.SemaphoreType.DMA,
            pltpu.SemaphoreType.DMA,
            pltpu.SemaphoreType.DMA,
            pltpu.SemaphoreType.DMA,
            pltpu.SemaphoreType.DMA,
            pltpu.SemaphoreType.DMA,
        ],
        compiler_params=pltpu.CompilerParams(use_tc_tiling_on_sc=False),
    )
    def sc_kernel(idx_hbm, table_hbm, pos_hbm, out_hbm,
                  idx_v, buf0, buf1, buf2, buf3, spos,
                  g0sem, g1sem, g2sem, g3sem,
                  p0sem, p1sem, p2sem, p3sem, wsem):
        sid = lax.axis_index("s")
        wid = sid * NC + lax.axis_index("c")
        wbase = wid * rows_per_w
        bufs = [buf0, buf1, buf2, buf3]
        gsems = [g0sem, g1sem, g2sem, g3sem]
        psems = [p0sem, p1sem, p2sem, p3sem]

        @pl.when(sid == 0)
        def _():
            pltpu.sync_copy(pos_hbm, spos)

        plsc.subcore_barrier()

        pltpu.sync_copy(idx_hbm.at[wid], idx_v)

        def prefill(r):
            return pltpu.async_copy(
                spos.at[pl.ds(0, chunk_rows)], bufs[r], psems[r])

        def writeback(c, r):
            return pltpu.async_copy(
                bufs[r],
                out_hbm.at[pl.ds(wbase + c * chunk_rows, chunk_rows),
                           pl.ds(0, D_MODEL)],
                wsem)

        gd = [None] * 4
        wd = [None] * 4
        pf = [None] * 4
        pf[0] = prefill(0)
        if n_chunks > 1:
            pf[1] = prefill(1)

        for c in range(n_chunks + 1):
            if c < n_chunks:
                r = c % 4
                pf[r].wait()
                gd[r] = pltpu.async_copy(
                    table_hbm.at[idx_v.at[c]], bufs[r], gsems[r], add=True)
            if c >= 1:
                rp = (c - 1) % 4
                gd[rp].wait()
                wd[rp] = writeback(c - 1, rp)
            if c + 2 <= n_chunks - 1:
                rn = (c + 2) % 4
                if wd[rn] is not None:
                    wd[rn].wait()
                pf[rn] = prefill(rn)

        for d in wd:
            if d is not None:
                d.wait()

    return sc_kernel


@jax.jit
def kernel(x, table):
    batch, seq_len = x.shape
    n_rows = batch * seq_len
    rows_per_w = n_rows // NW
    chunk_rows = 400
    n_chunks = rows_per_w // chunk_rows

    pos = jnp.asarray(
        np.tile(_pos_encoding_np(MAX_LEN, D_MODEL), (POS_REP, 1)))
    idx = x.reshape(NW, n_chunks, chunk_rows).astype(jnp.int32)

    sc_kernel = _make_sc_kernel(n_rows, chunk_rows, n_chunks)
    out = sc_kernel(idx, table, pos)
    return out[:, :D_MODEL].reshape(batch, seq_len, D_MODEL)
